# Initial kernel scaffold; baseline (speedup 1.0000x reference)
import jax
import jax.numpy as jnp
from jax import lax
from jax.experimental import pallas as pl
from jax.experimental.pallas import tpu as pltpu
from jax.experimental.pallas import tpu_sc as plsc

N_VOCAB = 100000
OUT_DIM = 128

_B = 4096
_S = 50
_TOTAL = _B * _S

_NC = 2
_NS = 16
_NW = _NC * _NS

_CH = 128
_PER_W = _TOTAL // _NW
_N_CHUNKS = _PER_W // _CH
_NBUF = 5
_N_GROUPS = _N_CHUNKS // _NBUF


def _body(ids_hbm, table_hbm, out_hbm, idx_v, rows_v, gsem, osem):
    wid = lax.axis_index("s") * _NC + lax.axis_index("c")
    idx_row0 = wid * _N_CHUNKS
    out_row0 = wid * _PER_W

    pltpu.sync_copy(ids_hbm.at[pl.ds(idx_row0, _N_CHUNKS)], idx_v)

    def gather_start(j, b):
        pltpu.make_async_copy(
            table_hbm.at[idx_v.at[j]], rows_v.at[b], gsem.at[b]
        ).start()

    def gather_wait(j, b):
        pltpu.make_async_copy(
            table_hbm.at[idx_v.at[j]], rows_v.at[b], gsem.at[b]
        ).wait()

    def out_copy(j, b):
        return pltpu.make_async_copy(
            rows_v.at[b], out_hbm.at[pl.ds(out_row0 + j * _CH, _CH)], osem.at[b]
        )

    for b in range(_NBUF):
        gather_start(b, b)

    def group(g_idx, carry):
        g = g_idx * _NBUF
        outs = []
        for b in range(_NBUF):
            j = g + b
            gather_wait(j, b)
            cp = out_copy(j, b)
            cp.start()
            outs.append((cp, j, b))
        for cp, j, b in outs:
            cp.wait()

            @pl.when(j + _NBUF < _N_CHUNKS)
            def _():
                gather_start(j + _NBUF, b)

        return carry

    lax.fori_loop(0, _N_GROUPS, group, 0)


@jax.jit
def _run(ids2d, table):
    mesh = plsc.VectorSubcoreMesh(core_axis_name="c", subcore_axis_name="s")
    return pl.kernel(
        _body,
        out_type=jax.ShapeDtypeStruct((_TOTAL, OUT_DIM), jnp.float32),
        mesh=mesh,
        scratch_types=[
            pltpu.VMEM((_N_CHUNKS, _CH), jnp.int32),
            pltpu.VMEM((_NBUF, _CH, OUT_DIM), jnp.float32),
            pltpu.SemaphoreType.DMA((_NBUF,)),
            pltpu.SemaphoreType.DMA((_NBUF,)),
        ],
    )(ids2d, table)


def kernel(input_ids, table):
    ids2d = input_ids.astype(jnp.int32).reshape(_TOTAL // _CH, _CH)
    out = _run(ids2d, table)
    return out.reshape(_B, _S, OUT_DIM)


# speedup vs baseline: 3.3117x; 3.3117x over previous
import jax
import jax.numpy as jnp
from jax import lax
from jax.experimental import pallas as pl
from jax.experimental.pallas import tpu as pltpu
from jax.experimental.pallas import tpu_sc as plsc

N_VOCAB = 100000
OUT_DIM = 128

_B = 4096
_S = 50
_TOTAL = _B * _S

_NC = 2
_NS = 16
_NW = _NC * _NS

_CH = 128
_PER_W = _TOTAL // _NW
_N_CHUNKS = _PER_W // _CH
_NBUF = 5
_N_GROUPS = _N_CHUNKS // _NBUF


def _body(ids_hbm, table_hbm, out_hbm, idx_v, rows_v, gsem, osem):
    wid = lax.axis_index("s") * _NC + lax.axis_index("c")
    out_row0 = wid * _PER_W

    pltpu.sync_copy(ids_hbm.at[wid], idx_v)

    def gather_start(j, b):
        pltpu.make_async_copy(
            table_hbm.at[idx_v.at[j]], rows_v.at[b], gsem.at[b]
        ).start()

    def gather_wait(j, b):
        pltpu.make_async_copy(
            table_hbm.at[idx_v.at[j]], rows_v.at[b], gsem.at[b]
        ).wait()

    def out_copy(j, b):
        return pltpu.make_async_copy(
            rows_v.at[b], out_hbm.at[pl.ds(out_row0 + j * _CH, _CH)], osem.at[b]
        )

    for b in range(_NBUF):
        gather_start(b, b)

    def group(g_idx, carry):
        g = g_idx * _NBUF
        outs = []
        for b in range(_NBUF):
            j = g + b
            gather_wait(j, b)
            cp = out_copy(j, b)
            cp.start()
            outs.append((cp, j, b))
        for cp, j, b in outs:
            cp.wait()

            @pl.when(j + _NBUF < _N_CHUNKS)
            def _():
                gather_start(j + _NBUF, b)

        return carry

    lax.fori_loop(0, _N_GROUPS, group, 0)


@jax.jit
def _run(ids3d, table):
    mesh = plsc.VectorSubcoreMesh(core_axis_name="c", subcore_axis_name="s")
    return pl.kernel(
        _body,
        out_type=jax.ShapeDtypeStruct((_TOTAL, OUT_DIM), jnp.float32),
        mesh=mesh,
        scratch_types=[
            pltpu.VMEM((_N_CHUNKS, _CH), jnp.int32),
            pltpu.VMEM((_NBUF, _CH, OUT_DIM), jnp.float32),
            pltpu.SemaphoreType.DMA((_NBUF,)),
            pltpu.SemaphoreType.DMA((_NBUF,)),
        ],
    )(ids3d, table)


def kernel(input_ids, table):
    ids3d = input_ids.astype(jnp.int32).reshape(_NW, _N_CHUNKS, _CH)
    out = _run(ids3d, table)
    return out.reshape(_B, _S, OUT_DIM)


# speedup vs baseline: 5.8692x; 1.7723x over previous
import jax
import jax.numpy as jnp
from jax import lax
from jax.experimental import pallas as pl
from jax.experimental.pallas import tpu as pltpu
from jax.experimental.pallas import tpu_sc as plsc

N_VOCAB = 100000
OUT_DIM = 128

_B = 4096
_S = 50

_NC = 2
_NS = 16
_NW = _NC * _NS

_SEQ_PER_W = _B // _NW
_CSEQ = 4
_N_CHUNKS = _SEQ_PER_W // _CSEQ
_NBUF = 4
_N_GROUPS = _N_CHUNKS // _NBUF


def _body(ids_hbm, table_hbm, out_hbm, idx_v, rows_v, gsem, osem):
    wid = lax.axis_index("s") * _NC + lax.axis_index("c")
    seq0 = wid * _SEQ_PER_W

    pltpu.sync_copy(ids_hbm.at[pl.ds(seq0, _SEQ_PER_W)], idx_v)

    def gather_start(j, b):
        for k in range(_CSEQ):
            pltpu.make_async_copy(
                table_hbm.at[idx_v.at[j * _CSEQ + k]],
                rows_v.at[b].at[k],
                gsem.at[b],
            ).start()

    def gather_wait(j, b):
        for k in range(_CSEQ):
            pltpu.make_async_copy(
                table_hbm.at[idx_v.at[j * _CSEQ + k]],
                rows_v.at[b].at[k],
                gsem.at[b],
            ).wait()

    def out_copy(j, b):
        return pltpu.make_async_copy(
            rows_v.at[b],
            out_hbm.at[pl.ds(seq0 + j * _CSEQ, _CSEQ)],
            osem.at[b],
        )

    for b in range(_NBUF):
        gather_start(b, b)

    def group(g_idx, carry):
        g = g_idx * _NBUF
        outs = []
        for b in range(_NBUF):
            j = g + b
            gather_wait(j, b)
            cp = out_copy(j, b)
            cp.start()
            outs.append((cp, j, b))
        for cp, j, b in outs:
            cp.wait()

            @pl.when(j + _NBUF < _N_CHUNKS)
            def _():
                gather_start(j + _NBUF, b)

        return carry

    lax.fori_loop(0, _N_GROUPS, group, 0)


@jax.jit
def _run(ids, table):
    mesh = plsc.VectorSubcoreMesh(core_axis_name="c", subcore_axis_name="s")
    return pl.kernel(
        _body,
        out_type=jax.ShapeDtypeStruct((_B, _S, OUT_DIM), jnp.float32),
        mesh=mesh,
        scratch_types=[
            pltpu.VMEM((_SEQ_PER_W, _S), jnp.int32),
            pltpu.VMEM((_NBUF, _CSEQ, _S, OUT_DIM), jnp.float32),
            pltpu.SemaphoreType.DMA((_NBUF,)),
            pltpu.SemaphoreType.DMA((_NBUF,)),
        ],
    )(ids, table)


def kernel(input_ids, table):
    return _run(input_ids.astype(jnp.int32), table)


# speedup vs baseline: 10.1587x; 1.7309x over previous
import jax
import jax.numpy as jnp
from jax import lax
from jax.experimental import pallas as pl
from jax.experimental.pallas import tpu as pltpu
from jax.experimental.pallas import tpu_sc as plsc

N_VOCAB = 100000
OUT_DIM = 128

_B = 4096
_S = 50
_TOTAL = _B * _S

_NC = 2
_NS = 16
_NW = _NC * _NS

_CH = 128
_PER_W = _TOTAL // _NW
_N_CHUNKS = _PER_W // _CH
_NBUF = 5
_N_GROUPS = _N_CHUNKS // _NBUF


def _body(ids_hbm, table_hbm, out_hbm, idx_v, rows_v, gsem, osem):
    wid = lax.axis_index("s") * _NC + lax.axis_index("c")
    out_row0 = wid * _PER_W

    pltpu.sync_copy(ids_hbm.at[wid], idx_v)

    def gather_start(j, b):
        pltpu.make_async_copy(
            table_hbm.at[idx_v.at[j]], rows_v.at[b], gsem.at[b]
        ).start()

    def gather_wait(j, b):
        pltpu.make_async_copy(
            table_hbm.at[idx_v.at[j]], rows_v.at[b], gsem.at[b]
        ).wait()

    def out_copy(j, b):
        return pltpu.make_async_copy(
            rows_v.at[b],
            out_hbm.at[pl.ds(out_row0 + j * _CH, _CH)],
            osem.at[b],
        )

    for b in range(_NBUF):
        gather_start(b, b)

    def group(g_idx, carry):
        g = g_idx * _NBUF
        outs = []
        for b in range(_NBUF):
            j = g + b
            gather_wait(j, b)
            cp = out_copy(j, b)
            cp.start()
            outs.append((cp, j, b))
        for cp, j, b in outs:
            cp.wait()

            @pl.when(j + _NBUF < _N_CHUNKS)
            def _():
                gather_start(j + _NBUF, b)

        return carry

    lax.fori_loop(0, _N_GROUPS, group, 0)


@jax.jit
def _run(ids3d, table):
    mesh = plsc.VectorSubcoreMesh(core_axis_name="c", subcore_axis_name="s")
    out = pl.kernel(
        _body,
        out_type=jax.ShapeDtypeStruct((_TOTAL, OUT_DIM), jnp.float32),
        mesh=mesh,
        scratch_types=[
            pltpu.VMEM((_N_CHUNKS, _CH), jnp.int32),
            pltpu.VMEM((_NBUF, _CH, OUT_DIM), jnp.float32),
            pltpu.SemaphoreType.DMA((_NBUF,)),
            pltpu.SemaphoreType.DMA((_NBUF,)),
        ],
    )(ids3d, table)
    return out.reshape(_S, _B, OUT_DIM).transpose(1, 0, 2)


def kernel(input_ids, table):
    ids_t = jnp.transpose(input_ids.astype(jnp.int32))
    ids3d = ids_t.reshape(_NW, _N_CHUNKS, _CH)
    return _run(ids3d, table)


# speedup vs baseline: 10.2543x; 1.0094x over previous
import jax
import jax.numpy as jnp
from jax import lax
from jax.experimental import pallas as pl
from jax.experimental.pallas import tpu as pltpu
from jax.experimental.pallas import tpu_sc as plsc

N_VOCAB = 100000
OUT_DIM = 128

_B = 4096
_S = 50
_TOTAL = _B * _S

_NC = 2
_NS = 16
_NW = _NC * _NS

_CH = 64
_PER_W = _TOTAL // _NW
_N_CHUNKS = _PER_W // _CH
_NBUF = 10
_N_GROUPS = _N_CHUNKS // _NBUF


def _body(ids_hbm, table_hbm, out_hbm, idx_v, rows_v, gsem, osem):
    wid = lax.axis_index("s") * _NC + lax.axis_index("c")
    out_row0 = wid * _PER_W

    pltpu.sync_copy(ids_hbm.at[wid], idx_v)

    def gather_start(j, b):
        pltpu.make_async_copy(
            table_hbm.at[idx_v.at[j]], rows_v.at[b], gsem.at[b]
        ).start()

    def gather_wait(j, b):
        pltpu.make_async_copy(
            table_hbm.at[idx_v.at[j]], rows_v.at[b], gsem.at[b]
        ).wait()

    def out_copy(j, b):
        return pltpu.make_async_copy(
            rows_v.at[b],
            out_hbm.at[pl.ds(out_row0 + j * _CH, _CH)],
            osem.at[b],
        )

    for b in range(_NBUF):
        gather_start(b, b)

    def group(g_idx, carry):
        g = g_idx * _NBUF
        outs = []
        for b in range(_NBUF):
            j = g + b
            gather_wait(j, b)
            cp = out_copy(j, b)
            cp.start()
            outs.append((cp, j, b))
        for cp, j, b in outs:
            cp.wait()

            @pl.when(j + _NBUF < _N_CHUNKS)
            def _():
                gather_start(j + _NBUF, b)

        return carry

    lax.fori_loop(0, _N_GROUPS, group, 0)


@jax.jit
def _run(ids3d, table):
    mesh = plsc.VectorSubcoreMesh(core_axis_name="c", subcore_axis_name="s")
    out = pl.kernel(
        _body,
        out_type=jax.ShapeDtypeStruct((_TOTAL, OUT_DIM), jnp.float32),
        mesh=mesh,
        scratch_types=[
            pltpu.VMEM((_N_CHUNKS, _CH), jnp.int32),
            pltpu.VMEM((_NBUF, _CH, OUT_DIM), jnp.float32),
            pltpu.SemaphoreType.DMA((_NBUF,)),
            pltpu.SemaphoreType.DMA((_NBUF,)),
        ],
    )(ids3d, table)
    return out.reshape(_S, _B, OUT_DIM).transpose(1, 0, 2)


def kernel(input_ids, table):
    ids_t = jnp.transpose(input_ids.astype(jnp.int32))
    ids3d = ids_t.reshape(_NW, _N_CHUNKS, _CH)
    return _run(ids3d, table)


# speedup vs baseline: 10.7124x; 1.0447x over previous
import jax
import jax.numpy as jnp
from jax import lax
from jax.experimental import pallas as pl
from jax.experimental.pallas import tpu as pltpu
from jax.experimental.pallas import tpu_sc as plsc

N_VOCAB = 100000
OUT_DIM = 128

_B = 4096
_S = 50
_TOTAL = _B * _S

_NC = 2
_NS = 16
_NW = _NC * _NS

_COLS = _B // _NW
_CH = 64
_PER_W = _S * _COLS
_N_CHUNKS = _PER_W // _CH
_NBUF = 10
_N_GROUPS = _N_CHUNKS // _NBUF
_CPS = _COLS // _CH


def _body(ids_hbm, table_hbm, out_hbm, idx_v, rows_v, gsem, osem):
    wid = lax.axis_index("s") * _NC + lax.axis_index("c")
    col0 = wid * _COLS

    pltpu.sync_copy(ids_hbm.at[:, pl.ds(col0, _COLS)], idx_v)

    def gather_start(j, b):
        pltpu.make_async_copy(
            table_hbm.at[idx_v.at[j // _CPS, pl.ds((j % _CPS) * _CH, _CH)]],
            rows_v.at[b],
            gsem.at[b],
        ).start()

    def gather_wait(j, b):
        pltpu.make_async_copy(
            table_hbm.at[idx_v.at[j // _CPS, pl.ds((j % _CPS) * _CH, _CH)]],
            rows_v.at[b],
            gsem.at[b],
        ).wait()

    def out_copy(j, b):
        row0 = (j // _CPS) * _B + col0 + (j % _CPS) * _CH
        return pltpu.make_async_copy(
            rows_v.at[b],
            out_hbm.at[pl.ds(row0, _CH)],
            osem.at[b],
        )

    for b in range(_NBUF):
        gather_start(b, b)

    def group(g_idx, carry):
        g = g_idx * _NBUF
        outs = []
        for b in range(_NBUF):
            j = g + b
            gather_wait(j, b)
            cp = out_copy(j, b)
            cp.start()
            outs.append((cp, j, b))
        for cp, j, b in outs:
            cp.wait()

            @pl.when(j + _NBUF < _N_CHUNKS)
            def _():
                gather_start(j + _NBUF, b)

        return carry

    lax.fori_loop(0, _N_GROUPS, group, 0)


@jax.jit
def _run(ids_t, table):
    mesh = plsc.VectorSubcoreMesh(core_axis_name="c", subcore_axis_name="s")
    out = pl.kernel(
        _body,
        out_type=jax.ShapeDtypeStruct((_TOTAL, OUT_DIM), jnp.float32),
        mesh=mesh,
        scratch_types=[
            pltpu.VMEM((_S, _COLS), jnp.int32),
            pltpu.VMEM((_NBUF, _CH, OUT_DIM), jnp.float32),
            pltpu.SemaphoreType.DMA((_NBUF,)),
            pltpu.SemaphoreType.DMA((_NBUF,)),
        ],
    )(ids_t, table)
    return out.reshape(_S, _B, OUT_DIM).transpose(1, 0, 2)


def kernel(input_ids, table):
    ids_t = jnp.transpose(input_ids.astype(jnp.int32))
    return _run(ids_t, table)
